# baseline (speedup 1.0000x reference)
import functools

import jax
import jax.numpy as jnp
from jax import lax
from jax.experimental import pallas as pl
from jax.experimental.pallas import tpu as pltpu
from jax.experimental.pallas import tpu_sc as plsc

_B, _P, _T = 32, 6375, 10
_L = 16
_PP = 6384
_NCH = _PP // _L
_NC, _NS = 2, 16
_LN2 = 0.6931471805599453


def _fastlog(x):
    b = lax.bitcast_convert_type(x, jnp.int32)
    e = jnp.right_shift(b, 23) - 127
    m = lax.bitcast_convert_type(
        jnp.bitwise_or(jnp.bitwise_and(b, 0x007FFFFF), 0x3F800000), jnp.float32)
    big = m > 1.3333334
    m = jnp.where(big, m * 0.5, m)
    e = (e + jnp.where(big, 1, 0)).astype(jnp.float32)
    s = (m - 1.0) / (m + 1.0)
    s2 = s * s
    p = 2.0 * s * (1.0 + s2 * (0.33333333 + s2 * (0.2 + s2 * 0.14285714)))
    return e * _LN2 + p


def _sc_body(pri_h, loc_h, conf_h, tgt_h, out_h,
             pri_v, loc_v, conf_v, tgt_v, bto_v, bti_v, rb_v, o_v):
    wid = lax.axis_index("s") * _NC + lax.axis_index("c")
    pltpu.sync_copy(pri_h, pri_v)
    pltpu.sync_copy(loc_h.at[wid], loc_v)
    pltpu.sync_copy(conf_h.at[wid], conf_v)
    pltpu.sync_copy(tgt_h.at[wid], tgt_v.at[pl.ds(0, 5)])

    iota = lax.iota(jnp.int32, _L)

    tx1v, ty1v, tx2v, ty2v = (tgt_v[1, :], tgt_v[2, :], tgt_v[3, :], tgt_v[4, :])
    tgt_v[5, :] = (tx2v - tx1v) * (ty2v - ty1v)

    def bc(row, t):
        return plsc.load_gather(
            tgt_v, [jnp.full((_L,), row, jnp.int32), jnp.full((_L,), t, jnp.int32)])

    def chunk1(c, carry):
        maxs, idxs = carry
        base = c * _L
        cx = pri_v[0, pl.ds(base, _L)]
        cy = pri_v[1, pl.ds(base, _L)]
        w = pri_v[2, pl.ds(base, _L)]
        h = pri_v[3, pl.ds(base, _L)]
        x1 = cx - 0.5 * w
        y1 = cy - 0.5 * h
        x2 = cx + 0.5 * w
        y2 = cy + 0.5 * h
        pa = w * h
        bto = jnp.full((_L,), -1.0, jnp.float32)
        bti = jnp.zeros((_L,), jnp.int32)
        maxs2, idxs2 = [], []
        for t in range(_T):
            iw = jnp.maximum(jnp.minimum(bc(3, t), x2) - jnp.maximum(bc(1, t), x1), 0.0)
            ih = jnp.maximum(jnp.minimum(bc(4, t), y2) - jnp.maximum(bc(2, t), y1), 0.0)
            inter = iw * ih
            ov = inter / (bc(5, t) + pa - inter)
            g = ov > bto
            bti = jnp.where(g, t, bti)
            bto = jnp.maximum(bto, ov)
            g2 = ov > maxs[t]
            idxs2.append(jnp.where(g2, base + iota, idxs[t]))
            maxs2.append(jnp.maximum(maxs[t], ov))
        bto_v[pl.ds(base, _L)] = bto
        bti_v[pl.ds(base, _L)] = bti
        return tuple(maxs2), tuple(idxs2)

    init = (tuple(jnp.full((_L,), -2.0, jnp.float32) for _ in range(_T)),
            tuple(jnp.zeros((_L,), jnp.int32) for _ in range(_T)))
    maxs, idxs = lax.fori_loop(0, _NCH, chunk1, init, unroll=3)

    bps = []
    for t in range(_T):
        m = jnp.max(maxs[t])
        bps.append(jnp.min(jnp.where(maxs[t] == m, idxs[t], jnp.int32(2**30))))

    def chunk2(c, carry):
        sl1a, cepa, npa = carry
        base = c * _L
        valid = (base + iota) < _P
        bto = bto_v[pl.ds(base, _L)]
        bti = bti_v[pl.ds(base, _L)]
        pidx = base + iota
        forced = jnp.zeros((_L,), jnp.bool_)
        for t in range(_T):
            mk = pidx == bps[t]
            bti = jnp.where(mk, t, bti)
            forced = jnp.logical_or(forced, mk)
        posv = jnp.logical_and(jnp.logical_or(bto >= 0.5, forced), valid)
        mx1 = plsc.load_gather(tgt_v, [jnp.full((_L,), 1, jnp.int32), bti])
        my1 = plsc.load_gather(tgt_v, [jnp.full((_L,), 2, jnp.int32), bti])
        mx2 = plsc.load_gather(tgt_v, [jnp.full((_L,), 3, jnp.int32), bti])
        my2 = plsc.load_gather(tgt_v, [jnp.full((_L,), 4, jnp.int32), bti])
        cx = pri_v[0, pl.ds(base, _L)]
        cy = pri_v[1, pl.ds(base, _L)]
        w = pri_v[2, pl.ds(base, _L)]
        h = pri_v[3, pl.ds(base, _L)]
        g0 = ((mx1 + mx2) * 0.5 - cx) / (0.1 * w)
        g1 = ((my1 + my2) * 0.5 - cy) / (0.1 * h)
        g2 = _fastlog((mx2 - mx1) / w) / 0.2
        g3 = _fastlog((my2 - my1) / h) / 0.2
        acc = sl1a
        pidx4 = pidx * 4
        pidx2 = pidx + pidx
        for j, g in enumerate((g0, g1, g2, g3)):
            lj = plsc.load_gather(loc_v, [pidx4 + j], mask=valid)
            d = lj - g
            ad = jnp.abs(d)
            e = jnp.where(ad < 1.0, 0.5 * d * d, ad - 0.5)
            acc = acc + jnp.where(posv, e, 0.0)
        c0 = plsc.load_gather(conf_v, [pidx2], mask=valid)
        c1 = plsc.load_gather(conf_v, [pidx2 + 1], mask=valid)
        dd = c1 - c0
        sp = _fastlog(1.0 + jnp.exp(-jnp.abs(dd)))
        r = jnp.maximum(dd, 0.0) + sp
        cep = jnp.maximum(-dd, 0.0) + sp
        cepa = cepa + jnp.where(posv, cep, 0.0)
        npa = npa + jnp.where(posv, 1.0, 0.0)
        rb = jnp.where(posv, 0, lax.bitcast_convert_type(r, jnp.int32))
        rb_v[pl.ds(base, _L)] = jnp.where(valid, rb, -1)
        return acc, cepa, npa

    z = jnp.zeros((_L,), jnp.float32)
    sl1a, cepa, npa = lax.fori_loop(0, _NCH, chunk2, (z, z, z), unroll=3)
    lsum = jnp.sum(sl1a)
    cepsum = jnp.sum(cepa)
    np_s = jnp.sum(npa)
    k = jnp.clip(3 * np_s.astype(jnp.int32), 2, _P - 1)

    def bs(i, lohi):
        lo, hi = lohi
        mid = lo + jnp.right_shift(hi - lo + 1, 1)

        def cnt_chunk(c, a):
            return a + jnp.where(rb_v[pl.ds(c * _L, _L)] >= mid, 1, 0)

        cnt = jnp.sum(lax.fori_loop(0, _NCH, cnt_chunk, jnp.zeros((_L,), jnp.int32), unroll=7))
        take = cnt >= k
        return jnp.where(take, mid, lo), jnp.where(take, hi, mid - 1)

    lo, _ = lax.fori_loop(0, 31, bs, (jnp.int32(0), jnp.int32(0x7F000000)))

    def chunk3(c, carry):
        ca, sa = carry
        rb = rb_v[pl.ds(c * _L, _L)]
        gt = rb > lo
        ca = ca + jnp.where(gt, 1, 0)
        sa = sa + jnp.where(gt, lax.bitcast_convert_type(rb, jnp.float32), 0.0)
        return ca, sa

    ca, sa = lax.fori_loop(0, _NCH, chunk3,
                           (jnp.zeros((_L,), jnp.int32), jnp.zeros((_L,), jnp.float32)),
                           unroll=7)
    c_gt = jnp.sum(ca)
    s_gt = jnp.sum(sa)
    tf = lax.bitcast_convert_type(lo, jnp.float32)
    tpos = lo > 0
    negsum = s_gt + jnp.where(tpos, (k - c_gt).astype(jnp.float32) * tf, 0.0)
    nb = np_s + jnp.where(tpos, k, c_gt).astype(jnp.float32)

    outv = jnp.where(iota == 0, lsum,
                     jnp.where(iota == 1, cepsum + negsum,
                               jnp.where(iota == 2, nb, 0.0)))
    o_v[...] = outv
    pltpu.sync_copy(o_v, out_h.at[wid])


@jax.jit
def kernel(arm_loc, arm_conf, odm_loc, odm_conf, priors, targets):
    del odm_loc, odm_conf
    pad = _PP - _P
    pri_t = jnp.concatenate(
        [priors.T, jnp.broadcast_to(jnp.array([-100.0, -100.0, 0.01, 0.01],
                                              jnp.float32)[:, None], (4, pad))], axis=1)
    tgt_t = jnp.pad(targets.transpose(0, 2, 1), ((0, 0), (0, 0), (0, _L - _T)))

    kern = functools.partial(
        pl.kernel,
        out_type=jax.ShapeDtypeStruct((_B, _L), jnp.float32),
        mesh=plsc.VectorSubcoreMesh(core_axis_name="c", subcore_axis_name="s",
                                    num_cores=_NC, num_subcores=_NS),
        scratch_types=[
            pltpu.VMEM((4, _PP), jnp.float32),
            pltpu.VMEM((_P * 4,), jnp.float32),
            pltpu.VMEM((_P * 2,), jnp.float32),
            pltpu.VMEM((6, _L), jnp.float32),
            pltpu.VMEM((_PP,), jnp.float32),
            pltpu.VMEM((_PP,), jnp.int32),
            pltpu.VMEM((_PP,), jnp.int32),
            pltpu.VMEM((_L,), jnp.float32),
        ],
        compiler_params=pltpu.CompilerParams(needs_layout_passes=False),
    )(_sc_body)
    parts = kern(pri_t, arm_loc.reshape(_B, _P * 4),
                 arm_conf.reshape(_B, _P * 2), tgt_t)
    n = jnp.sum(parts[:, 2])
    return jnp.sum(parts[:, 0]) / n, jnp.sum(parts[:, 1]) / n


# speedup vs baseline: 1.4309x; 1.4309x over previous
import functools

import jax
import jax.numpy as jnp
from jax import lax
from jax.experimental import pallas as pl
from jax.experimental.pallas import tpu as pltpu
from jax.experimental.pallas import tpu_sc as plsc

_B, _P, _T = 32, 6375, 10
_L = 16
_PP = 6384
_NCH = _PP // _L
_NC, _NS = 2, 16
_LN2 = 0.6931471805599453


def _fastlog(x):
    b = lax.bitcast_convert_type(x, jnp.int32)
    e = jnp.right_shift(b, 23) - 127
    m = lax.bitcast_convert_type(
        jnp.bitwise_or(jnp.bitwise_and(b, 0x007FFFFF), 0x3F800000), jnp.float32)
    big = m > 1.3333334
    m = jnp.where(big, m * 0.5, m)
    e = (e + jnp.where(big, 1, 0)).astype(jnp.float32)
    s = (m - 1.0) / (m + 1.0)
    s2 = s * s
    p = 2.0 * s * (1.0 + s2 * (0.33333333 + s2 * (0.2 + s2 * 0.14285714)))
    return e * _LN2 + p


def _sc_body(pri_h, loc_h, conf_h, tgt_h, out_h,
             pri_v, loc_v, conf_v, tgt_v, bto_v, bti_v, rb_v, o_v):
    wid = lax.axis_index("s") * _NC + lax.axis_index("c")
    pltpu.sync_copy(pri_h, pri_v)
    pltpu.sync_copy(loc_h.at[wid], loc_v)
    pltpu.sync_copy(conf_h.at[wid], conf_v)
    pltpu.sync_copy(tgt_h.at[wid], tgt_v.at[pl.ds(0, 5)])

    iota = lax.iota(jnp.int32, _L)

    tx1v, ty1v, tx2v, ty2v = (tgt_v[1, :], tgt_v[2, :], tgt_v[3, :], tgt_v[4, :])
    tgt_v[5, :] = (tx2v - tx1v) * (ty2v - ty1v)

    def bc(row, t):
        return plsc.load_gather(
            tgt_v, [jnp.full((_L,), row, jnp.int32), jnp.full((_L,), t, jnp.int32)])

    def chunk1(c, carry):
        maxs, idxs = carry
        base = c * _L
        cx = pri_v[0, pl.ds(base, _L)]
        cy = pri_v[1, pl.ds(base, _L)]
        w = pri_v[2, pl.ds(base, _L)]
        h = pri_v[3, pl.ds(base, _L)]
        x1 = cx - 0.5 * w
        y1 = cy - 0.5 * h
        x2 = cx + 0.5 * w
        y2 = cy + 0.5 * h
        pa = w * h
        bto = jnp.full((_L,), -1.0, jnp.float32)
        bti = jnp.zeros((_L,), jnp.int32)
        maxs2, idxs2 = [], []
        for t in range(_T):
            iw = jnp.maximum(jnp.minimum(bc(3, t), x2) - jnp.maximum(bc(1, t), x1), 0.0)
            ih = jnp.maximum(jnp.minimum(bc(4, t), y2) - jnp.maximum(bc(2, t), y1), 0.0)
            inter = iw * ih
            ov = inter / (bc(5, t) + pa - inter)
            g = ov > bto
            bti = jnp.where(g, t, bti)
            bto = jnp.maximum(bto, ov)
            g2 = ov > maxs[t]
            idxs2.append(jnp.where(g2, base + iota, idxs[t]))
            maxs2.append(jnp.maximum(maxs[t], ov))
        bto_v[pl.ds(base, _L)] = bto
        bti_v[pl.ds(base, _L)] = bti
        return tuple(maxs2), tuple(idxs2)

    init = (tuple(jnp.full((_L,), -2.0, jnp.float32) for _ in range(_T)),
            tuple(jnp.zeros((_L,), jnp.int32) for _ in range(_T)))
    maxs, idxs = lax.fori_loop(0, _NCH, chunk1, init, unroll=3)

    bps = []
    for t in range(_T):
        m = jnp.max(maxs[t])
        bps.append(jnp.min(jnp.where(maxs[t] == m, idxs[t], jnp.int32(2**30))))

    def chunk2(c, carry):
        sl1a, cepa, npa = carry
        base = c * _L
        valid = (base + iota) < _P
        bto = bto_v[pl.ds(base, _L)]
        bti = bti_v[pl.ds(base, _L)]
        pidx = base + iota
        forced = jnp.zeros((_L,), jnp.bool_)
        for t in range(_T):
            mk = pidx == bps[t]
            bti = jnp.where(mk, t, bti)
            forced = jnp.logical_or(forced, mk)
        posv = jnp.logical_and(jnp.logical_or(bto >= 0.5, forced), valid)
        mx1 = plsc.load_gather(tgt_v, [jnp.full((_L,), 1, jnp.int32), bti])
        my1 = plsc.load_gather(tgt_v, [jnp.full((_L,), 2, jnp.int32), bti])
        mx2 = plsc.load_gather(tgt_v, [jnp.full((_L,), 3, jnp.int32), bti])
        my2 = plsc.load_gather(tgt_v, [jnp.full((_L,), 4, jnp.int32), bti])
        cx = pri_v[0, pl.ds(base, _L)]
        cy = pri_v[1, pl.ds(base, _L)]
        w = pri_v[2, pl.ds(base, _L)]
        h = pri_v[3, pl.ds(base, _L)]
        g0 = ((mx1 + mx2) * 0.5 - cx) / (0.1 * w)
        g1 = ((my1 + my2) * 0.5 - cy) / (0.1 * h)
        g2 = _fastlog((mx2 - mx1) / w) / 0.2
        g3 = _fastlog((my2 - my1) / h) / 0.2
        acc = sl1a
        for j, g in enumerate((g0, g1, g2, g3)):
            d = loc_v[j, pl.ds(base, _L)] - g
            ad = jnp.abs(d)
            e = jnp.where(ad < 1.0, 0.5 * d * d, ad - 0.5)
            acc = acc + jnp.where(posv, e, 0.0)
        c0 = conf_v[0, pl.ds(base, _L)]
        c1 = conf_v[1, pl.ds(base, _L)]
        dd = c1 - c0
        sp = _fastlog(1.0 + jnp.exp(-jnp.abs(dd)))
        r = jnp.maximum(dd, 0.0) + sp
        cep = jnp.maximum(-dd, 0.0) + sp
        cepa = cepa + jnp.where(posv, cep, 0.0)
        npa = npa + jnp.where(posv, 1.0, 0.0)
        rb = jnp.where(posv, 0, lax.bitcast_convert_type(r, jnp.int32))
        rb_v[pl.ds(base, _L)] = jnp.where(valid, rb, -1)
        return acc, cepa, npa

    z = jnp.zeros((_L,), jnp.float32)
    sl1a, cepa, npa = lax.fori_loop(0, _NCH, chunk2, (z, z, z), unroll=3)
    lsum = jnp.sum(sl1a)
    cepsum = jnp.sum(cepa)
    np_s = jnp.sum(npa)
    k = jnp.clip(3 * np_s.astype(jnp.int32), 2, _P - 1)

    def bs(i, lohi):
        lo, hi = lohi
        mid = lo + jnp.right_shift(hi - lo + 1, 1)

        def cnt_chunk(c, a):
            return a + jnp.where(rb_v[pl.ds(c * _L, _L)] >= mid, 1, 0)

        cnt = jnp.sum(lax.fori_loop(0, _NCH, cnt_chunk, jnp.zeros((_L,), jnp.int32), unroll=7))
        take = cnt >= k
        return jnp.where(take, mid, lo), jnp.where(take, hi, mid - 1)

    lo, _ = lax.fori_loop(0, 31, bs, (jnp.int32(0), jnp.int32(0x7F000000)))

    def chunk3(c, carry):
        ca, sa = carry
        rb = rb_v[pl.ds(c * _L, _L)]
        gt = rb > lo
        ca = ca + jnp.where(gt, 1, 0)
        sa = sa + jnp.where(gt, lax.bitcast_convert_type(rb, jnp.float32), 0.0)
        return ca, sa

    ca, sa = lax.fori_loop(0, _NCH, chunk3,
                           (jnp.zeros((_L,), jnp.int32), jnp.zeros((_L,), jnp.float32)),
                           unroll=7)
    c_gt = jnp.sum(ca)
    s_gt = jnp.sum(sa)
    tf = lax.bitcast_convert_type(lo, jnp.float32)
    tpos = lo > 0
    negsum = s_gt + jnp.where(tpos, (k - c_gt).astype(jnp.float32) * tf, 0.0)
    nb = np_s + jnp.where(tpos, k, c_gt).astype(jnp.float32)

    outv = jnp.where(iota == 0, lsum,
                     jnp.where(iota == 1, cepsum + negsum,
                               jnp.where(iota == 2, nb, 0.0)))
    o_v[...] = outv
    pltpu.sync_copy(o_v, out_h.at[wid])


@jax.jit
def kernel(arm_loc, arm_conf, odm_loc, odm_conf, priors, targets):
    del odm_loc, odm_conf
    pad = _PP - _P
    pri_t = jnp.concatenate(
        [priors.T, jnp.broadcast_to(jnp.array([-100.0, -100.0, 0.01, 0.01],
                                              jnp.float32)[:, None], (4, pad))], axis=1)
    tgt_t = jnp.pad(targets.transpose(0, 2, 1), ((0, 0), (0, 0), (0, _L - _T)))

    kern = functools.partial(
        pl.kernel,
        out_type=jax.ShapeDtypeStruct((_B, _L), jnp.float32),
        mesh=plsc.VectorSubcoreMesh(core_axis_name="c", subcore_axis_name="s",
                                    num_cores=_NC, num_subcores=_NS),
        scratch_types=[
            pltpu.VMEM((4, _PP), jnp.float32),
            pltpu.VMEM((4, _PP), jnp.float32),
            pltpu.VMEM((2, _PP), jnp.float32),
            pltpu.VMEM((6, _L), jnp.float32),
            pltpu.VMEM((_PP,), jnp.float32),
            pltpu.VMEM((_PP,), jnp.int32),
            pltpu.VMEM((_PP,), jnp.int32),
            pltpu.VMEM((_L,), jnp.float32),
        ],
        compiler_params=pltpu.CompilerParams(needs_layout_passes=False),
    )(_sc_body)
    loc_t = jnp.pad(arm_loc.transpose(0, 2, 1), ((0, 0), (0, 0), (0, pad)))
    conf_t = jnp.pad(arm_conf.transpose(0, 2, 1), ((0, 0), (0, 0), (0, pad)))
    parts = kern(pri_t, loc_t, conf_t, tgt_t)
    n = jnp.sum(parts[:, 2])
    return jnp.sum(parts[:, 0]) / n, jnp.sum(parts[:, 1]) / n


# speedup vs baseline: 1.4651x; 1.0239x over previous
import functools

import jax
import jax.numpy as jnp
from jax import lax
from jax.experimental import pallas as pl
from jax.experimental.pallas import tpu as pltpu
from jax.experimental.pallas import tpu_sc as plsc

_B, _P, _T = 32, 6375, 10
_L = 16
_PP = 6384
_NCH = _PP // _L
_NC, _NS = 2, 16
_LN2 = 0.6931471805599453


def _fastlog(x):
    b = lax.bitcast_convert_type(x, jnp.int32)
    e = jnp.right_shift(b, 23) - 127
    m = lax.bitcast_convert_type(
        jnp.bitwise_or(jnp.bitwise_and(b, 0x007FFFFF), 0x3F800000), jnp.float32)
    big = m > 1.3333334
    m = jnp.where(big, m * 0.5, m)
    e = (e + jnp.where(big, 1, 0)).astype(jnp.float32)
    s = (m - 1.0) / (m + 1.0)
    s2 = s * s
    p = 2.0 * s * (1.0 + s2 * (0.33333333 + s2 * (0.2 + s2 * 0.14285714)))
    return e * _LN2 + p


def _sc_body(pri_h, loc_h, conf_h, tgt_h, out_h,
             pri_v, loc_v, conf_v, tgt_v, bto_v, bti_v, rb_v, o_v, sem):
    wid = lax.axis_index("s") * _NC + lax.axis_index("c")
    cp_loc = pltpu.async_copy(loc_h.at[wid], loc_v, sem)
    cp_conf = pltpu.async_copy(conf_h.at[wid], conf_v, sem)
    pltpu.sync_copy(pri_h, pri_v)
    pltpu.sync_copy(tgt_h.at[wid], tgt_v.at[pl.ds(0, 5)])

    iota = lax.iota(jnp.int32, _L)

    tx1v, ty1v, tx2v, ty2v = (tgt_v[1, :], tgt_v[2, :], tgt_v[3, :], tgt_v[4, :])
    tgt_v[5, :] = (tx2v - tx1v) * (ty2v - ty1v)

    def bc(row, t):
        return plsc.load_gather(
            tgt_v, [jnp.full((_L,), row, jnp.int32), jnp.full((_L,), t, jnp.int32)])

    def chunk1(c, carry):
        maxs, idxs = carry
        base = c * _L
        cx = pri_v[0, pl.ds(base, _L)]
        cy = pri_v[1, pl.ds(base, _L)]
        w = pri_v[2, pl.ds(base, _L)]
        h = pri_v[3, pl.ds(base, _L)]
        x1 = cx - 0.5 * w
        y1 = cy - 0.5 * h
        x2 = cx + 0.5 * w
        y2 = cy + 0.5 * h
        pa = w * h
        bto = jnp.full((_L,), -1.0, jnp.float32)
        bti = jnp.zeros((_L,), jnp.int32)
        maxs2, idxs2 = [], []
        for t in range(_T):
            iw = jnp.maximum(jnp.minimum(bc(3, t), x2) - jnp.maximum(bc(1, t), x1), 0.0)
            ih = jnp.maximum(jnp.minimum(bc(4, t), y2) - jnp.maximum(bc(2, t), y1), 0.0)
            inter = iw * ih
            ov = inter / (bc(5, t) + pa - inter)
            g = ov > bto
            bti = jnp.where(g, t, bti)
            bto = jnp.maximum(bto, ov)
            g2 = ov > maxs[t]
            idxs2.append(jnp.where(g2, base + iota, idxs[t]))
            maxs2.append(jnp.maximum(maxs[t], ov))
        bto_v[pl.ds(base, _L)] = bto
        bti_v[pl.ds(base, _L)] = bti
        return tuple(maxs2), tuple(idxs2)

    init = (tuple(jnp.full((_L,), -2.0, jnp.float32) for _ in range(_T)),
            tuple(jnp.zeros((_L,), jnp.int32) for _ in range(_T)))
    maxs, idxs = lax.fori_loop(0, _NCH, chunk1, init, unroll=3)

    bps = []
    for t in range(_T):
        m = jnp.max(maxs[t])
        bps.append(jnp.min(jnp.where(maxs[t] == m, idxs[t], jnp.int32(2**30))))
    bps_vec = jnp.zeros((_L,), jnp.int32)
    for t in range(_T):
        bps_vec = jnp.where(iota == t, bps[t], bps_vec)
    eff = jnp.zeros((_L,), jnp.int32)
    for t in range(_T):
        eff = jnp.where(bps_vec == bps[t], t, eff)
    lane_ok = iota < _T
    plsc.store_scatter(bto_v, [bps_vec], jnp.full((_L,), 2.0, jnp.float32),
                       mask=lane_ok)
    plsc.store_scatter(bti_v, [bps_vec], eff, mask=lane_ok)
    cp_loc.wait()
    cp_conf.wait()

    def chunk2(c, carry):
        sl1a, cepa, npa = carry
        base = c * _L
        valid = (base + iota) < _P
        bto = bto_v[pl.ds(base, _L)]
        posv = jnp.logical_and(bto >= 0.5, valid)
        c0 = conf_v[0, pl.ds(base, _L)]
        c1 = conf_v[1, pl.ds(base, _L)]
        dd = c1 - c0
        sp = _fastlog(1.0 + jnp.exp(-jnp.abs(dd)))
        r = jnp.maximum(dd, 0.0) + sp
        rb = jnp.where(posv, 0, lax.bitcast_convert_type(r, jnp.int32))
        rb_v[pl.ds(base, _L)] = jnp.where(valid, rb, -1)
        npa = npa + jnp.where(posv, 1.0, 0.0)

        def with_pos(ops):
            acc, cepa = ops
            bti = bti_v[pl.ds(base, _L)]
            mx1 = plsc.load_gather(tgt_v, [jnp.full((_L,), 1, jnp.int32), bti])
            my1 = plsc.load_gather(tgt_v, [jnp.full((_L,), 2, jnp.int32), bti])
            mx2 = plsc.load_gather(tgt_v, [jnp.full((_L,), 3, jnp.int32), bti])
            my2 = plsc.load_gather(tgt_v, [jnp.full((_L,), 4, jnp.int32), bti])
            cx = pri_v[0, pl.ds(base, _L)]
            cy = pri_v[1, pl.ds(base, _L)]
            w = pri_v[2, pl.ds(base, _L)]
            h = pri_v[3, pl.ds(base, _L)]
            g0 = ((mx1 + mx2) * 0.5 - cx) / (0.1 * w)
            g1 = ((my1 + my2) * 0.5 - cy) / (0.1 * h)
            g2 = _fastlog((mx2 - mx1) / w) / 0.2
            g3 = _fastlog((my2 - my1) / h) / 0.2
            for j, g in enumerate((g0, g1, g2, g3)):
                d = loc_v[j, pl.ds(base, _L)] - g
                ad = jnp.abs(d)
                e = jnp.where(ad < 1.0, 0.5 * d * d, ad - 0.5)
                acc = acc + jnp.where(posv, e, 0.0)
            cep = jnp.maximum(-dd, 0.0) + sp
            return acc, cepa + jnp.where(posv, cep, 0.0)

        sl1a, cepa = lax.cond(jnp.any(posv), with_pos, lambda o: o, (sl1a, cepa))
        return sl1a, cepa, npa

    z = jnp.zeros((_L,), jnp.float32)
    sl1a, cepa, npa = lax.fori_loop(0, _NCH, chunk2, (z, z, z), unroll=3)
    lsum = jnp.sum(sl1a)
    cepsum = jnp.sum(cepa)
    np_s = jnp.sum(npa)
    k = jnp.clip(3 * np_s.astype(jnp.int32), 2, _P - 1)

    def bs(i, lohi):
        lo, hi = lohi
        mid = lo + jnp.right_shift(hi - lo + 1, 1)

        def cnt_chunk(c, a):
            return a + jnp.where(rb_v[pl.ds(c * _L, _L)] >= mid, 1, 0)

        cnt = jnp.sum(lax.fori_loop(0, _NCH, cnt_chunk, jnp.zeros((_L,), jnp.int32), unroll=7))
        take = cnt >= k
        return jnp.where(take, mid, lo), jnp.where(take, hi, mid - 1)

    lo, _ = lax.fori_loop(0, 31, bs, (jnp.int32(0), jnp.int32(0x7F000000)))

    def chunk3(c, carry):
        ca, sa = carry
        rb = rb_v[pl.ds(c * _L, _L)]
        gt = rb > lo
        ca = ca + jnp.where(gt, 1, 0)
        sa = sa + jnp.where(gt, lax.bitcast_convert_type(rb, jnp.float32), 0.0)
        return ca, sa

    ca, sa = lax.fori_loop(0, _NCH, chunk3,
                           (jnp.zeros((_L,), jnp.int32), jnp.zeros((_L,), jnp.float32)),
                           unroll=7)
    c_gt = jnp.sum(ca)
    s_gt = jnp.sum(sa)
    tf = lax.bitcast_convert_type(lo, jnp.float32)
    tpos = lo > 0
    negsum = s_gt + jnp.where(tpos, (k - c_gt).astype(jnp.float32) * tf, 0.0)
    nb = np_s + jnp.where(tpos, k, c_gt).astype(jnp.float32)

    outv = jnp.where(iota == 0, lsum,
                     jnp.where(iota == 1, cepsum + negsum,
                               jnp.where(iota == 2, nb, 0.0)))
    o_v[...] = outv
    pltpu.sync_copy(o_v, out_h.at[wid])


@jax.jit
def kernel(arm_loc, arm_conf, odm_loc, odm_conf, priors, targets):
    del odm_loc, odm_conf
    pad = _PP - _P
    pri_t = jnp.concatenate(
        [priors.T, jnp.broadcast_to(jnp.array([-100.0, -100.0, 0.01, 0.01],
                                              jnp.float32)[:, None], (4, pad))], axis=1)
    tgt_t = jnp.pad(targets.transpose(0, 2, 1), ((0, 0), (0, 0), (0, _L - _T)))

    kern = functools.partial(
        pl.kernel,
        out_type=jax.ShapeDtypeStruct((_B, _L), jnp.float32),
        mesh=plsc.VectorSubcoreMesh(core_axis_name="c", subcore_axis_name="s",
                                    num_cores=_NC, num_subcores=_NS),
        scratch_types=[
            pltpu.VMEM((4, _PP), jnp.float32),
            pltpu.VMEM((4, _PP), jnp.float32),
            pltpu.VMEM((2, _PP), jnp.float32),
            pltpu.VMEM((6, _L), jnp.float32),
            pltpu.VMEM((_PP,), jnp.float32),
            pltpu.VMEM((_PP,), jnp.int32),
            pltpu.VMEM((_PP,), jnp.int32),
            pltpu.VMEM((_L,), jnp.float32),
            pltpu.SemaphoreType.DMA,
        ],
        compiler_params=pltpu.CompilerParams(needs_layout_passes=False),
    )(_sc_body)
    loc_t = jnp.pad(arm_loc.transpose(0, 2, 1), ((0, 0), (0, 0), (0, pad)))
    conf_t = jnp.pad(arm_conf.transpose(0, 2, 1), ((0, 0), (0, 0), (0, pad)))
    parts = kern(pri_t, loc_t, conf_t, tgt_t)
    n = jnp.sum(parts[:, 2])
    return jnp.sum(parts[:, 0]) / n, jnp.sum(parts[:, 1]) / n


# speedup vs baseline: 1.6175x; 1.1040x over previous
import functools

import jax
import jax.numpy as jnp
from jax import lax
from jax.experimental import pallas as pl
from jax.experimental.pallas import tpu as pltpu
from jax.experimental.pallas import tpu_sc as plsc

_B, _P, _T = 32, 6375, 10
_L = 16
_PP = 6384
_NCH = _PP // _L
_NC, _NS = 2, 16
_LN2 = 0.6931471805599453


def _fastlog(x):
    b = lax.bitcast_convert_type(x, jnp.int32)
    e = jnp.right_shift(b, 23) - 127
    m = lax.bitcast_convert_type(
        jnp.bitwise_or(jnp.bitwise_and(b, 0x007FFFFF), 0x3F800000), jnp.float32)
    big = m > 1.3333334
    m = jnp.where(big, m * 0.5, m)
    e = (e + jnp.where(big, 1, 0)).astype(jnp.float32)
    s = (m - 1.0) / (m + 1.0)
    s2 = s * s
    p = 2.0 * s * (1.0 + s2 * (0.33333333 + s2 * (0.2 + s2 * 0.14285714)))
    return e * _LN2 + p


def _sc_body(pri_h, loc_h, conf_h, tgt_h, out_h,
             pri_v, loc_v, conf_v, tgt_v, bto_v, bti_v, rb_v, o_v, sem):
    wid = lax.axis_index("s") * _NC + lax.axis_index("c")
    cp_loc = pltpu.async_copy(loc_h.at[wid], loc_v, sem)
    cp_conf = pltpu.async_copy(conf_h.at[wid], conf_v, sem)
    pltpu.sync_copy(pri_h, pri_v)
    pltpu.sync_copy(tgt_h.at[wid], tgt_v.at[pl.ds(0, 5)])

    iota = lax.iota(jnp.int32, _L)

    tx1v, ty1v, tx2v, ty2v = (tgt_v[1, :], tgt_v[2, :], tgt_v[3, :], tgt_v[4, :])
    tgt_v[5, :] = (tx2v - tx1v) * (ty2v - ty1v)

    def bc(row, t):
        return plsc.load_gather(
            tgt_v, [jnp.full((_L,), row, jnp.int32), jnp.full((_L,), t, jnp.int32)])

    tb = [(bc(1, t), bc(2, t), bc(3, t), bc(4, t), bc(5, t)) for t in range(_T)]

    def chunk1(c, carry):
        maxs, idxs = carry
        base = c * _L
        cx = pri_v[0, pl.ds(base, _L)]
        cy = pri_v[1, pl.ds(base, _L)]
        w = pri_v[2, pl.ds(base, _L)]
        h = pri_v[3, pl.ds(base, _L)]
        x1 = cx - 0.5 * w
        y1 = cy - 0.5 * h
        x2 = cx + 0.5 * w
        y2 = cy + 0.5 * h
        pa = w * h
        bto = jnp.full((_L,), -1.0, jnp.float32)
        bti = jnp.zeros((_L,), jnp.int32)
        maxs2, idxs2 = [], []
        for t in range(_T):
            tx1, ty1, tx2, ty2, ta = tb[t]
            iw = jnp.maximum(jnp.minimum(tx2, x2) - jnp.maximum(tx1, x1), 0.0)
            ih = jnp.maximum(jnp.minimum(ty2, y2) - jnp.maximum(ty1, y1), 0.0)
            inter = iw * ih
            ov = inter / (ta + pa - inter)
            g = ov > bto
            bti = jnp.where(g, t, bti)
            bto = jnp.maximum(bto, ov)
            g2 = ov > maxs[t]
            idxs2.append(jnp.where(g2, base + iota, idxs[t]))
            maxs2.append(jnp.maximum(maxs[t], ov))
        bto_v[pl.ds(base, _L)] = bto
        bti_v[pl.ds(base, _L)] = bti
        return tuple(maxs2), tuple(idxs2)

    init = (tuple(jnp.full((_L,), -2.0, jnp.float32) for _ in range(_T)),
            tuple(jnp.zeros((_L,), jnp.int32) for _ in range(_T)))
    maxs, idxs = lax.fori_loop(0, _NCH, chunk1, init, unroll=3)

    bps = []
    for t in range(_T):
        m = jnp.max(maxs[t])
        bps.append(jnp.min(jnp.where(maxs[t] == m, idxs[t], jnp.int32(2**30))))
    bps_vec = jnp.zeros((_L,), jnp.int32)
    for t in range(_T):
        bps_vec = jnp.where(iota == t, bps[t], bps_vec)
    eff = jnp.zeros((_L,), jnp.int32)
    for t in range(_T):
        eff = jnp.where(bps_vec == bps[t], t, eff)
    lane_ok = iota < _T
    plsc.store_scatter(bto_v, [bps_vec], jnp.full((_L,), 2.0, jnp.float32),
                       mask=lane_ok)
    plsc.store_scatter(bti_v, [bps_vec], eff, mask=lane_ok)
    cp_loc.wait()
    cp_conf.wait()

    def chunk2(c, carry):
        sl1a, cepa, npa = carry
        base = c * _L
        valid = (base + iota) < _P
        bto = bto_v[pl.ds(base, _L)]
        posv = jnp.logical_and(bto >= 0.5, valid)
        c0 = conf_v[0, pl.ds(base, _L)]
        c1 = conf_v[1, pl.ds(base, _L)]
        dd = c1 - c0
        sp = _fastlog(1.0 + jnp.exp(-jnp.abs(dd)))
        r = jnp.maximum(dd, 0.0) + sp
        rb = jnp.where(posv, 0, lax.bitcast_convert_type(r, jnp.int32))
        rb_v[pl.ds(base, _L)] = jnp.where(valid, rb, -1)
        npa = npa + jnp.where(posv, 1.0, 0.0)

        def with_pos(ops):
            acc, cepa = ops
            bti = bti_v[pl.ds(base, _L)]
            mx1 = plsc.load_gather(tgt_v, [jnp.full((_L,), 1, jnp.int32), bti])
            my1 = plsc.load_gather(tgt_v, [jnp.full((_L,), 2, jnp.int32), bti])
            mx2 = plsc.load_gather(tgt_v, [jnp.full((_L,), 3, jnp.int32), bti])
            my2 = plsc.load_gather(tgt_v, [jnp.full((_L,), 4, jnp.int32), bti])
            cx = pri_v[0, pl.ds(base, _L)]
            cy = pri_v[1, pl.ds(base, _L)]
            w = pri_v[2, pl.ds(base, _L)]
            h = pri_v[3, pl.ds(base, _L)]
            g0 = ((mx1 + mx2) * 0.5 - cx) / (0.1 * w)
            g1 = ((my1 + my2) * 0.5 - cy) / (0.1 * h)
            g2 = _fastlog((mx2 - mx1) / w) / 0.2
            g3 = _fastlog((my2 - my1) / h) / 0.2
            for j, g in enumerate((g0, g1, g2, g3)):
                d = loc_v[j, pl.ds(base, _L)] - g
                ad = jnp.abs(d)
                e = jnp.where(ad < 1.0, 0.5 * d * d, ad - 0.5)
                acc = acc + jnp.where(posv, e, 0.0)
            cep = jnp.maximum(-dd, 0.0) + sp
            return acc, cepa + jnp.where(posv, cep, 0.0)

        sl1a, cepa = lax.cond(jnp.any(posv), with_pos, lambda o: o, (sl1a, cepa))
        return sl1a, cepa, npa

    z = jnp.zeros((_L,), jnp.float32)
    sl1a, cepa, npa = lax.fori_loop(0, _NCH, chunk2, (z, z, z), unroll=3)
    lsum = jnp.sum(sl1a)
    cepsum = jnp.sum(cepa)
    np_s = jnp.sum(npa)
    k = jnp.clip(3 * np_s.astype(jnp.int32), 2, _P - 1)

    def bs(i, lohi):
        lo, hi = lohi
        mid = lo + jnp.right_shift(hi - lo + 1, 1)

        def cnt_chunk(c, a):
            return a + jnp.where(rb_v[pl.ds(c * _L, _L)] >= mid, 1, 0)

        cnt = jnp.sum(lax.fori_loop(0, _NCH, cnt_chunk, jnp.zeros((_L,), jnp.int32), unroll=7))
        take = cnt >= k
        return jnp.where(take, mid, lo), jnp.where(take, hi, mid - 1)

    lo, _ = lax.fori_loop(0, 31, bs, (jnp.int32(0), jnp.int32(0x7F000000)))

    def chunk3(c, carry):
        ca, sa = carry
        rb = rb_v[pl.ds(c * _L, _L)]
        gt = rb > lo
        ca = ca + jnp.where(gt, 1, 0)
        sa = sa + jnp.where(gt, lax.bitcast_convert_type(rb, jnp.float32), 0.0)
        return ca, sa

    ca, sa = lax.fori_loop(0, _NCH, chunk3,
                           (jnp.zeros((_L,), jnp.int32), jnp.zeros((_L,), jnp.float32)),
                           unroll=7)
    c_gt = jnp.sum(ca)
    s_gt = jnp.sum(sa)
    tf = lax.bitcast_convert_type(lo, jnp.float32)
    tpos = lo > 0
    negsum = s_gt + jnp.where(tpos, (k - c_gt).astype(jnp.float32) * tf, 0.0)
    nb = np_s + jnp.where(tpos, k, c_gt).astype(jnp.float32)

    outv = jnp.where(iota == 0, lsum,
                     jnp.where(iota == 1, cepsum + negsum,
                               jnp.where(iota == 2, nb, 0.0)))
    o_v[...] = outv
    pltpu.sync_copy(o_v, out_h.at[wid])


@jax.jit
def kernel(arm_loc, arm_conf, odm_loc, odm_conf, priors, targets):
    del odm_loc, odm_conf
    pad = _PP - _P
    pri_t = jnp.concatenate(
        [priors.T, jnp.broadcast_to(jnp.array([-100.0, -100.0, 0.01, 0.01],
                                              jnp.float32)[:, None], (4, pad))], axis=1)
    tgt_t = jnp.pad(targets.transpose(0, 2, 1), ((0, 0), (0, 0), (0, _L - _T)))

    kern = functools.partial(
        pl.kernel,
        out_type=jax.ShapeDtypeStruct((_B, _L), jnp.float32),
        mesh=plsc.VectorSubcoreMesh(core_axis_name="c", subcore_axis_name="s",
                                    num_cores=_NC, num_subcores=_NS),
        scratch_types=[
            pltpu.VMEM((4, _PP), jnp.float32),
            pltpu.VMEM((4, _PP), jnp.float32),
            pltpu.VMEM((2, _PP), jnp.float32),
            pltpu.VMEM((6, _L), jnp.float32),
            pltpu.VMEM((_PP,), jnp.float32),
            pltpu.VMEM((_PP,), jnp.int32),
            pltpu.VMEM((_PP,), jnp.int32),
            pltpu.VMEM((_L,), jnp.float32),
            pltpu.SemaphoreType.DMA,
        ],
        compiler_params=pltpu.CompilerParams(needs_layout_passes=False),
    )(_sc_body)
    loc_t = jnp.pad(arm_loc.transpose(0, 2, 1), ((0, 0), (0, 0), (0, pad)))
    conf_t = jnp.pad(arm_conf.transpose(0, 2, 1), ((0, 0), (0, 0), (0, pad)))
    parts = kern(pri_t, loc_t, conf_t, tgt_t)
    n = jnp.sum(parts[:, 2])
    return jnp.sum(parts[:, 0]) / n, jnp.sum(parts[:, 1]) / n
